# baseline (speedup 1.0000x reference)
import functools

import jax
import jax.numpy as jnp
from jax import lax
from jax.experimental import pallas as pl
from jax.experimental.pallas import tpu as pltpu
from jax.experimental.pallas import tpu_sc as plsc

N = 16 * 512 * 512
K = max(int(N * (1.0 - 0.7)), max(1, 10000))
NB = 1024
SHIFT1, SHIFT2 = 21, 11
NW = 32
PER_W = N // NW
CH = 16384
NCH = PER_W // CH
HB = NB * 16


def _loss_body(x_ref, t_ref, o_ref):
    x = x_ref[...]
    t = t_ref[...]
    o_ref[...] = jnp.maximum(x, 0.0) - x * t + jnp.log1p(jnp.exp(-jnp.abs(x)))


def _loss(x2, t2):
    rows, cols = x2.shape
    blk = 512
    return pl.pallas_call(
        _loss_body,
        grid=(rows // blk,),
        in_specs=[pl.BlockSpec((blk, cols), lambda i: (i, 0))] * 2,
        out_specs=pl.BlockSpec((blk, cols), lambda i: (i, 0)),
        out_shape=jax.ShapeDtypeStruct((rows, cols), jnp.float32),
    )(x2, t2)


@functools.cache
def _make_sc_pass(prefix_shift, digit_shift, self_select):
    mesh = plsc.VectorSubcoreMesh(
        core_axis_name="c", subcore_axis_name="s", num_cores=2, num_subcores=16)
    nbank = 1 if self_select else 2

    @functools.partial(
        pl.kernel,
        out_type=(
            jax.ShapeDtypeStruct((NW, NB), jnp.int32),
            jax.ShapeDtypeStruct((NW, NB), jnp.float32),
        ),
        mesh=mesh,
        scratch_types=[
            pltpu.VMEM((CH,), jnp.float32),
            pltpu.VMEM((CH,), jnp.float32),
            pltpu.VMEM((nbank * HB,), jnp.int32),
            pltpu.VMEM((nbank * HB,), jnp.float32),
            pltpu.VMEM((NB,), jnp.int32),
            pltpu.VMEM((NB,), jnp.float32),
            pltpu.VMEM((NW, NB) if self_select else (16,), jnp.int32),
            pltpu.SemaphoreType.DMA,
            pltpu.SemaphoreType.DMA,
        ],
        compiler_params=pltpu.CompilerParams(needs_layout_passes=False),
    )
    def sc_pass(loss_hbm, sel_hbm, cnt_out, sum_out,
                buf0, buf1, cnt_h, sum_h, red_c, red_s, selv, sem0, sem1):
        wid = lax.axis_index("s") * 2 + lax.axis_index("c")
        base = wid * PER_W
        lane = lax.iota(jnp.int32, 16)
        pltpu.sync_copy(sel_hbm, selv)
        if not self_select:
            sel = selv[...]
        else:
            def srow(cidx, carry):
                sl = pl.ds(cidx * 16, 16)
                p = [selv[w, sl] for w in range(4)]
                for w in range(4, NW):
                    p[w % 4] += selv[w, sl]
                red_c[sl] = (p[0] + p[1]) + (p[2] + p[3])
                return carry

            lax.fori_loop(0, NB // 16, srow, 0)
            kvec = jnp.full((16,), K, jnp.int32)

            def sscan(i, carry):
                m_acc, csum = carry
                vc = red_c[pl.ds((NB // 16 - 1 - i) * 16, 16)]
                cs = plsc.cumsum(lax.rev(vc, (0,))) + csum
                m_acc = m_acc + plsc.all_reduce_population_count(cs >= kvec)
                csum = jnp.broadcast_to(jnp.max(cs), (16,))
                return m_acc, csum

            m_acc, _ = lax.fori_loop(
                0, NB // 16, sscan,
                (jnp.zeros((16,), jnp.int32), jnp.zeros((16,), jnp.int32)))
            sel = m_acc - 1

        zi = jnp.zeros((16,), jnp.int32)
        zf = jnp.zeros((16,), jnp.float32)

        def zero_body(j, carry):
            for u in range(8):
                cnt_h[pl.ds(j * 128 + u * 16, 16)] = zi
                sum_h[pl.ds(j * 128 + u * 16, 16)] = zf
            return carry

        lax.fori_loop(0, nbank * HB // 128, zero_body, 0)

        ones = jnp.ones((16,), jnp.int32)
        bufs, sems = (buf0, buf1), (sem0, sem1)
        copies = [None, None]
        copies[0] = pltpu.async_copy(loss_hbm.at[pl.ds(base, CH)], buf0, sem0)
        for g in range(NCH):
            if g + 1 < NCH:
                copies[(g + 1) % 2] = pltpu.async_copy(
                    loss_hbm.at[pl.ds(base + (g + 1) * CH, CH)],
                    bufs[(g + 1) % 2], sems[(g + 1) % 2])
            copies[g % 2].wait()
            buf = bufs[g % 2]

            def step(j, carry):
                vs, idxs, ms = [], [], []
                for u in range(8):
                    v = buf[pl.ds(j * 128 + u * 16, 16)]
                    b = plsc.bitcast(v, jnp.int32)
                    m = lax.shift_right_logical(b, prefix_shift) == sel
                    digit = lax.shift_right_logical(b, digit_shift) & (NB - 1)
                    vs.append(v)
                    ms.append(m)
                    idxs.append(digit * 16 + lane + (u % nbank) * HB)
                for u in range(8):
                    plsc.addupdate_scatter(cnt_h, [idxs[u]], ones, mask=ms[u])
                    plsc.addupdate_scatter(sum_h, [idxs[u]], vs[u], mask=ms[u])
                return carry

            lax.fori_loop(0, CH // 128, step, 0)

        def red_body(cidx, carry):
            dig0 = (cidx * 16 + lane) * 16
            offs = [b * HB + l for b in range(nbank) for l in range(16)]
            pc = [plsc.load_gather(cnt_h, [dig0 + offs[p]]) for p in range(4)]
            ps = [plsc.load_gather(sum_h, [dig0 + offs[p]]) for p in range(4)]
            for i in range(4, len(offs)):
                pc[i % 4] += plsc.load_gather(cnt_h, [dig0 + offs[i]])
                ps[i % 4] += plsc.load_gather(sum_h, [dig0 + offs[i]])
            red_c[pl.ds(cidx * 16, 16)] = (pc[0] + pc[1]) + (pc[2] + pc[3])
            red_s[pl.ds(cidx * 16, 16)] = (ps[0] + ps[1]) + (ps[2] + ps[3])
            return carry

        lax.fori_loop(0, NB // 16, red_body, 0)
        pltpu.sync_copy(red_c, cnt_out.at[wid])
        pltpu.sync_copy(red_s, sum_out.at[wid])

    return sc_pass


def _sc_pass1(loss, sel):
    return _make_sc_pass(31, SHIFT1, False)(loss, sel)


def _sc_pass2(loss, cnt1):
    return _make_sc_pass(SHIFT1, SHIFT2, True)(loss, cnt1)


def _suffix_counts(cg):
    ii = lax.broadcasted_iota(jnp.int32, (NB, NB), 0)
    jj = lax.broadcasted_iota(jnp.int32, (NB, NB), 1)
    return jnp.sum(jnp.where(jj >= ii, cg[None, :], 0), axis=1)


def _sel2_body(c1_ref, s1_ref, c2_ref, s2_ref, out_ref):
    io = lax.iota(jnp.int32, NB)
    cg1 = jnp.sum(c1_ref[...], axis=0)
    sg1 = jnp.sum(s1_ref[...], axis=0)
    suffix1 = _suffix_counts(cg1)
    d1 = jnp.sum((suffix1 >= K).astype(jnp.int32)) - 1
    c1 = jnp.sum(jnp.where(io > d1, cg1, 0))
    s1 = jnp.sum(jnp.where(io > d1, sg1, 0.0))
    k2 = K - c1
    cg = jnp.sum(c2_ref[...], axis=0)
    sg = jnp.sum(s2_ref[...], axis=0)
    suffix = _suffix_counts(cg)
    d2 = jnp.sum((suffix >= k2).astype(jnp.int32)) - 1
    c2 = jnp.sum(jnp.where(io > d2, cg, 0))
    s2 = jnp.sum(jnp.where(io > d2, sg, 0.0))
    nb = jnp.sum(jnp.where(io == d2, cg, 0))
    sb = jnp.sum(jnp.where(io == d2, sg, 0.0))
    r = (k2 - c2).astype(jnp.float32)
    ans = (s1 + s2 + r * (sb / nb.astype(jnp.float32))) * (1.0 / K)
    out_ref[...] = jnp.full((8, 128), ans, jnp.float32)


def _sel2(c1, s1, c2, s2):
    return pl.pallas_call(
        _sel2_body,
        out_shape=jax.ShapeDtypeStruct((8, 128), jnp.float32),
    )(c1, s1, c2, s2)


def kernel(pred, target):
    x2 = pred.reshape(4096, 1024)
    t2 = target.reshape(4096, 1024)
    loss = _loss(x2, t2).reshape(N)
    zsel = jnp.zeros((16,), jnp.int32)
    cnt1, sum1 = _sc_pass1(loss, zsel)
    cnt2, sum2 = _sc_pass2(loss, cnt1)
    out = _sel2(cnt1, sum1, cnt2, sum2)
    return out[0, 0]


# speedup vs baseline: 1.1143x; 1.1143x over previous
import functools

import jax
import jax.numpy as jnp
from jax import lax
from jax.experimental import pallas as pl
from jax.experimental.pallas import tpu as pltpu
from jax.experimental.pallas import tpu_sc as plsc

N = 16 * 512 * 512
K = max(int(N * (1.0 - 0.7)), max(1, 10000))
NB = 1024
SHIFT1, SHIFT2 = 21, 11
NW = 32
PER_W = N // NW
CH = 16384
NCH = PER_W // CH
HB = NB * 16


def _loss_body(x_ref, t_ref, o_ref):
    x = x_ref[...]
    t = t_ref[...]
    loss = jnp.maximum(x, 0.0) - x * t + jnp.log1p(jnp.exp(-jnp.abs(x)))
    o_ref[...] = loss.reshape(o_ref.shape)


def _loss(x2, t2):
    rows, cols = x2.shape
    blk = 512
    return pl.pallas_call(
        _loss_body,
        grid=(rows // blk,),
        in_specs=[pl.BlockSpec((blk, cols), lambda i: (i, 0))] * 2,
        out_specs=pl.BlockSpec((blk * cols,), lambda i: (i,)),
        out_shape=jax.ShapeDtypeStruct((rows * cols,), jnp.float32),
    )(x2, t2)


@functools.cache
def _make_sc_pass(prefix_shift, digit_shift, self_select):
    mesh = plsc.VectorSubcoreMesh(
        core_axis_name="c", subcore_axis_name="s", num_cores=2, num_subcores=16)
    nbank = 1 if self_select else 2

    @functools.partial(
        pl.kernel,
        out_type=(
            jax.ShapeDtypeStruct((NW, NB), jnp.int32),
            jax.ShapeDtypeStruct((NW, NB), jnp.float32),
        ),
        mesh=mesh,
        scratch_types=[
            pltpu.VMEM((CH,), jnp.float32),
            pltpu.VMEM((CH,), jnp.float32),
            pltpu.VMEM((nbank * HB,), jnp.int32),
            pltpu.VMEM((nbank * HB,), jnp.float32),
            pltpu.VMEM((NB,), jnp.int32),
            pltpu.VMEM((NB,), jnp.float32),
            pltpu.VMEM((NW, NB) if self_select else (16,), jnp.int32),
            pltpu.SemaphoreType.DMA,
            pltpu.SemaphoreType.DMA,
        ],
        compiler_params=pltpu.CompilerParams(needs_layout_passes=False),
    )
    def sc_pass(loss_hbm, sel_hbm, cnt_out, sum_out,
                buf0, buf1, cnt_h, sum_h, red_c, red_s, selv, sem0, sem1):
        wid = lax.axis_index("s") * 2 + lax.axis_index("c")
        base = wid * PER_W
        lane = lax.iota(jnp.int32, 16)
        pltpu.sync_copy(sel_hbm, selv)
        if not self_select:
            sel = selv[...]
        else:
            def srow(cidx, carry):
                sl = pl.ds(cidx * 16, 16)
                p = [selv[w, sl] for w in range(4)]
                for w in range(4, NW):
                    p[w % 4] += selv[w, sl]
                red_c[sl] = (p[0] + p[1]) + (p[2] + p[3])
                return carry

            lax.fori_loop(0, NB // 16, srow, 0)
            kvec = jnp.full((16,), K, jnp.int32)

            def sscan(i, carry):
                m_acc, csum = carry
                vc = red_c[pl.ds((NB // 16 - 1 - i) * 16, 16)]
                cs = plsc.cumsum(lax.rev(vc, (0,))) + csum
                m_acc = m_acc + plsc.all_reduce_population_count(cs >= kvec)
                csum = jnp.broadcast_to(jnp.max(cs), (16,))
                return m_acc, csum

            m_acc, _ = lax.fori_loop(
                0, NB // 16, sscan,
                (jnp.zeros((16,), jnp.int32), jnp.zeros((16,), jnp.int32)))
            sel = m_acc - 1

        zi = jnp.zeros((16,), jnp.int32)
        zf = jnp.zeros((16,), jnp.float32)

        def zero_body(j, carry):
            for u in range(8):
                cnt_h[pl.ds(j * 128 + u * 16, 16)] = zi
                sum_h[pl.ds(j * 128 + u * 16, 16)] = zf
            return carry

        lax.fori_loop(0, nbank * HB // 128, zero_body, 0)

        ones = jnp.ones((16,), jnp.int32)
        bufs, sems = (buf0, buf1), (sem0, sem1)
        copies = [None, None]
        copies[0] = pltpu.async_copy(loss_hbm.at[pl.ds(base, CH)], buf0, sem0)
        for g in range(NCH):
            if g + 1 < NCH:
                copies[(g + 1) % 2] = pltpu.async_copy(
                    loss_hbm.at[pl.ds(base + (g + 1) * CH, CH)],
                    bufs[(g + 1) % 2], sems[(g + 1) % 2])
            copies[g % 2].wait()
            buf = bufs[g % 2]

            def step(j, carry):
                vs, idxs, ms = [], [], []
                for u in range(8):
                    v = buf[pl.ds(j * 128 + u * 16, 16)]
                    b = plsc.bitcast(v, jnp.int32)
                    m = lax.shift_right_logical(b, prefix_shift) == sel
                    digit = lax.shift_right_logical(b, digit_shift) & (NB - 1)
                    vs.append(v)
                    ms.append(m)
                    idxs.append(digit * 16 + lane + (u % nbank) * HB)
                for u in range(8):
                    plsc.addupdate_scatter(cnt_h, [idxs[u]], ones, mask=ms[u])
                    plsc.addupdate_scatter(sum_h, [idxs[u]], vs[u], mask=ms[u])
                return carry

            lax.fori_loop(0, CH // 128, step, 0)

        def red_body(cidx, carry):
            dig0 = (cidx * 16 + lane) * 16
            offs = [b * HB + l for b in range(nbank) for l in range(16)]
            pc = [plsc.load_gather(cnt_h, [dig0 + offs[p]]) for p in range(4)]
            ps = [plsc.load_gather(sum_h, [dig0 + offs[p]]) for p in range(4)]
            for i in range(4, len(offs)):
                pc[i % 4] += plsc.load_gather(cnt_h, [dig0 + offs[i]])
                ps[i % 4] += plsc.load_gather(sum_h, [dig0 + offs[i]])
            red_c[pl.ds(cidx * 16, 16)] = (pc[0] + pc[1]) + (pc[2] + pc[3])
            red_s[pl.ds(cidx * 16, 16)] = (ps[0] + ps[1]) + (ps[2] + ps[3])
            return carry

        lax.fori_loop(0, NB // 16, red_body, 0)
        pltpu.sync_copy(red_c, cnt_out.at[wid])
        pltpu.sync_copy(red_s, sum_out.at[wid])

    return sc_pass


def _sc_pass1(loss, sel):
    return _make_sc_pass(31, SHIFT1, False)(loss, sel)


def _sc_pass2(loss, cnt1):
    return _make_sc_pass(SHIFT1, SHIFT2, True)(loss, cnt1)


def _suffix_counts(cg):
    ii = lax.broadcasted_iota(jnp.int32, (NB, NB), 0)
    jj = lax.broadcasted_iota(jnp.int32, (NB, NB), 1)
    return jnp.sum(jnp.where(jj >= ii, cg[None, :], 0), axis=1)


def _sel2_body(c1_ref, s1_ref, c2_ref, s2_ref, out_ref):
    io = lax.iota(jnp.int32, NB)
    cg1 = jnp.sum(c1_ref[...], axis=0)
    sg1 = jnp.sum(s1_ref[...], axis=0)
    suffix1 = _suffix_counts(cg1)
    d1 = jnp.sum((suffix1 >= K).astype(jnp.int32)) - 1
    c1 = jnp.sum(jnp.where(io > d1, cg1, 0))
    s1 = jnp.sum(jnp.where(io > d1, sg1, 0.0))
    k2 = K - c1
    cg = jnp.sum(c2_ref[...], axis=0)
    sg = jnp.sum(s2_ref[...], axis=0)
    suffix = _suffix_counts(cg)
    d2 = jnp.sum((suffix >= k2).astype(jnp.int32)) - 1
    c2 = jnp.sum(jnp.where(io > d2, cg, 0))
    s2 = jnp.sum(jnp.where(io > d2, sg, 0.0))
    nb = jnp.sum(jnp.where(io == d2, cg, 0))
    sb = jnp.sum(jnp.where(io == d2, sg, 0.0))
    r = (k2 - c2).astype(jnp.float32)
    ans = (s1 + s2 + r * (sb / nb.astype(jnp.float32))) * (1.0 / K)
    out_ref[...] = jnp.full((8, 128), ans, jnp.float32)


def _sel2(c1, s1, c2, s2):
    return pl.pallas_call(
        _sel2_body,
        out_shape=jax.ShapeDtypeStruct((8, 128), jnp.float32),
    )(c1, s1, c2, s2)


def kernel(pred, target):
    x2 = pred.reshape(4096, 1024)
    t2 = target.reshape(4096, 1024)
    loss = _loss(x2, t2)
    zsel = jnp.zeros((16,), jnp.int32)
    cnt1, sum1 = _sc_pass1(loss, zsel)
    cnt2, sum2 = _sc_pass2(loss, cnt1)
    out = _sel2(cnt1, sum1, cnt2, sum2)
    return out[0, 0]


# speedup vs baseline: 1.5025x; 1.3483x over previous
import functools

import jax
import jax.numpy as jnp
from jax import lax
from jax.experimental import pallas as pl
from jax.experimental.pallas import tpu as pltpu
from jax.experimental.pallas import tpu_sc as plsc

N = 16 * 512 * 512
K = max(int(N * (1.0 - 0.7)), max(1, 10000))
NB = 1024
SHIFT1, SHIFT2 = 21, 11
NW = 32
PER_W = N // NW
CH = 16384
NCH = PER_W // CH
HB = NB * 16


def _loss_body(x_ref, t_ref, o_ref):
    blk = x_ref.shape[0]
    hw = x_ref.shape[2] * x_ref.shape[3]
    for bi in range(blk):
        x = x_ref[bi, 0]
        t = t_ref[bi, 0]
        loss = jnp.maximum(x, 0.0) - x * t + jnp.log1p(jnp.exp(-jnp.abs(x)))
        o_ref[pl.ds(bi * hw, hw)] = loss.reshape(hw)


def _loss(x4, t4):
    b, _, h, w = x4.shape
    blk = 2
    return pl.pallas_call(
        _loss_body,
        grid=(b // blk,),
        in_specs=[pl.BlockSpec((blk, 1, h, w), lambda i: (i, 0, 0, 0))] * 2,
        out_specs=pl.BlockSpec((blk * h * w,), lambda i: (i,)),
        out_shape=jax.ShapeDtypeStruct((b * h * w,), jnp.float32),
    )(x4, t4)


@functools.cache
def _make_sc_pass(prefix_shift, digit_shift, self_select):
    mesh = plsc.VectorSubcoreMesh(
        core_axis_name="c", subcore_axis_name="s", num_cores=2, num_subcores=16)
    nbank = 1 if self_select else 2

    @functools.partial(
        pl.kernel,
        out_type=(
            jax.ShapeDtypeStruct((NW, NB), jnp.int32),
            jax.ShapeDtypeStruct((NW, NB), jnp.float32),
        ),
        mesh=mesh,
        scratch_types=[
            pltpu.VMEM((CH,), jnp.float32),
            pltpu.VMEM((CH,), jnp.float32),
            pltpu.VMEM((nbank * HB,), jnp.int32),
            pltpu.VMEM((nbank * HB,), jnp.float32),
            pltpu.VMEM((NB,), jnp.int32),
            pltpu.VMEM((NB,), jnp.float32),
            pltpu.VMEM((NW, NB) if self_select else (16,), jnp.int32),
            pltpu.SemaphoreType.DMA,
            pltpu.SemaphoreType.DMA,
        ],
        compiler_params=pltpu.CompilerParams(needs_layout_passes=False),
    )
    def sc_pass(loss_hbm, sel_hbm, cnt_out, sum_out,
                buf0, buf1, cnt_h, sum_h, red_c, red_s, selv, sem0, sem1):
        wid = lax.axis_index("s") * 2 + lax.axis_index("c")
        base = wid * PER_W
        lane = lax.iota(jnp.int32, 16)
        pltpu.sync_copy(sel_hbm, selv)
        if not self_select:
            sel = selv[...]
        else:
            def srow(cidx, carry):
                sl = pl.ds(cidx * 16, 16)
                p = [selv[w, sl] for w in range(4)]
                for w in range(4, NW):
                    p[w % 4] += selv[w, sl]
                red_c[sl] = (p[0] + p[1]) + (p[2] + p[3])
                return carry

            lax.fori_loop(0, NB // 16, srow, 0)
            kvec = jnp.full((16,), K, jnp.int32)

            def sscan(i, carry):
                m_acc, csum = carry
                vc = red_c[pl.ds((NB // 16 - 1 - i) * 16, 16)]
                cs = plsc.cumsum(lax.rev(vc, (0,))) + csum
                m_acc = m_acc + plsc.all_reduce_population_count(cs >= kvec)
                csum = jnp.broadcast_to(jnp.max(cs), (16,))
                return m_acc, csum

            m_acc, _ = lax.fori_loop(
                0, NB // 16, sscan,
                (jnp.zeros((16,), jnp.int32), jnp.zeros((16,), jnp.int32)))
            sel = m_acc - 1

        zi = jnp.zeros((16,), jnp.int32)
        zf = jnp.zeros((16,), jnp.float32)

        def zero_body(j, carry):
            for u in range(8):
                cnt_h[pl.ds(j * 128 + u * 16, 16)] = zi
                sum_h[pl.ds(j * 128 + u * 16, 16)] = zf
            return carry

        lax.fori_loop(0, nbank * HB // 128, zero_body, 0)

        ones = jnp.ones((16,), jnp.int32)
        bufs, sems = (buf0, buf1), (sem0, sem1)
        copies = [None, None]
        copies[0] = pltpu.async_copy(loss_hbm.at[pl.ds(base, CH)], buf0, sem0)
        for g in range(NCH):
            if g + 1 < NCH:
                copies[(g + 1) % 2] = pltpu.async_copy(
                    loss_hbm.at[pl.ds(base + (g + 1) * CH, CH)],
                    bufs[(g + 1) % 2], sems[(g + 1) % 2])
            copies[g % 2].wait()
            buf = bufs[g % 2]

            def step(j, carry):
                vs, idxs, ms = [], [], []
                for u in range(8):
                    v = buf[pl.ds(j * 128 + u * 16, 16)]
                    b = plsc.bitcast(v, jnp.int32)
                    m = lax.shift_right_logical(b, prefix_shift) == sel
                    digit = lax.shift_right_logical(b, digit_shift) & (NB - 1)
                    vs.append(v)
                    ms.append(m)
                    idxs.append(digit * 16 + lane + (u % nbank) * HB)
                for u in range(8):
                    plsc.addupdate_scatter(cnt_h, [idxs[u]], ones, mask=ms[u])
                    plsc.addupdate_scatter(sum_h, [idxs[u]], vs[u], mask=ms[u])
                return carry

            lax.fori_loop(0, CH // 128, step, 0)

        def red_body(cidx, carry):
            dig0 = (cidx * 16 + lane) * 16
            offs = [b * HB + l for b in range(nbank) for l in range(16)]
            pc = [plsc.load_gather(cnt_h, [dig0 + offs[p]]) for p in range(4)]
            ps = [plsc.load_gather(sum_h, [dig0 + offs[p]]) for p in range(4)]
            for i in range(4, len(offs)):
                pc[i % 4] += plsc.load_gather(cnt_h, [dig0 + offs[i]])
                ps[i % 4] += plsc.load_gather(sum_h, [dig0 + offs[i]])
            red_c[pl.ds(cidx * 16, 16)] = (pc[0] + pc[1]) + (pc[2] + pc[3])
            red_s[pl.ds(cidx * 16, 16)] = (ps[0] + ps[1]) + (ps[2] + ps[3])
            return carry

        lax.fori_loop(0, NB // 16, red_body, 0)
        pltpu.sync_copy(red_c, cnt_out.at[wid])
        pltpu.sync_copy(red_s, sum_out.at[wid])

    return sc_pass


def _sc_pass1(loss, sel):
    return _make_sc_pass(31, SHIFT1, False)(loss, sel)


def _sc_pass2(loss, cnt1):
    return _make_sc_pass(SHIFT1, SHIFT2, True)(loss, cnt1)


def _suffix_counts(cg):
    ii = lax.broadcasted_iota(jnp.int32, (NB, NB), 0)
    jj = lax.broadcasted_iota(jnp.int32, (NB, NB), 1)
    return jnp.sum(jnp.where(jj >= ii, cg[None, :], 0), axis=1)


def _sel2_body(c1_ref, s1_ref, c2_ref, s2_ref, out_ref):
    io = lax.iota(jnp.int32, NB)
    cg1 = jnp.sum(c1_ref[...], axis=0)
    sg1 = jnp.sum(s1_ref[...], axis=0)
    suffix1 = _suffix_counts(cg1)
    d1 = jnp.sum((suffix1 >= K).astype(jnp.int32)) - 1
    c1 = jnp.sum(jnp.where(io > d1, cg1, 0))
    s1 = jnp.sum(jnp.where(io > d1, sg1, 0.0))
    k2 = K - c1
    cg = jnp.sum(c2_ref[...], axis=0)
    sg = jnp.sum(s2_ref[...], axis=0)
    suffix = _suffix_counts(cg)
    d2 = jnp.sum((suffix >= k2).astype(jnp.int32)) - 1
    c2 = jnp.sum(jnp.where(io > d2, cg, 0))
    s2 = jnp.sum(jnp.where(io > d2, sg, 0.0))
    nb = jnp.sum(jnp.where(io == d2, cg, 0))
    sb = jnp.sum(jnp.where(io == d2, sg, 0.0))
    r = (k2 - c2).astype(jnp.float32)
    ans = (s1 + s2 + r * (sb / nb.astype(jnp.float32))) * (1.0 / K)
    out_ref[...] = jnp.full((8, 128), ans, jnp.float32)


def _sel2(c1, s1, c2, s2):
    return pl.pallas_call(
        _sel2_body,
        out_shape=jax.ShapeDtypeStruct((8, 128), jnp.float32),
    )(c1, s1, c2, s2)


def kernel(pred, target):
    loss = _loss(pred, target)
    zsel = jnp.zeros((16,), jnp.int32)
    cnt1, sum1 = _sc_pass1(loss, zsel)
    cnt2, sum2 = _sc_pass2(loss, cnt1)
    out = _sel2(cnt1, sum1, cnt2, sum2)
    return out[0, 0]
